# Initial kernel scaffold; baseline (speedup 1.0000x reference)
import jax
import jax.numpy as jnp
from jax.experimental import pallas as pl


def kernel(x, edge_index, W1, b1, W2, b2):
    raise NotImplementedError("write your pallas kernel here")


# speedup vs baseline: 26.6796x; 26.6796x over previous
import functools

import jax
import jax.numpy as jnp
from jax import lax
from jax.experimental import pallas as pl
from jax.experimental.pallas import tpu as pltpu
from jax.experimental.pallas import tpu_sc as plsc

N = 10000
NP = 10240
E = 320000
DF = 128
DH = 256
DC = 64

NC = 2
NS = 16
NW = NC * NS
EPW = E // NW
CHUNK = 125
NCHUNK = EPW // CHUNK
RPT = NP // NS
DEGW = 16

_sc_mesh = plsc.VectorSubcoreMesh(core_axis_name="c", subcore_axis_name="s")


def _wid(c, s):
    return c * NS + s


def _deg_body(dst_hbm, ones_hbm, zeros_hbm, deg_out, idx_v, ones_v, sem, accum):
    c = lax.axis_index("c")
    s = lax.axis_index("s")
    rows = pl.ds(s * RPT, RPT)
    pltpu.sync_copy(zeros_hbm.at[rows], accum.at[rows])
    pltpu.sync_copy(ones_hbm, ones_v)
    pltpu.sync_copy(dst_hbm.at[_wid(c, s)], idx_v)
    plsc.subcore_barrier()

    def body(j, carry):
        pltpu.sync_copy(ones_v, accum.at[idx_v.at[j]], add=True)
        return carry

    lax.fori_loop(0, NCHUNK, body, 0)
    plsc.subcore_barrier()
    pltpu.sync_copy(accum.at[rows], deg_out.at[c, rows])


_deg_kernel = functools.partial(
    pl.kernel,
    out_type=jax.ShapeDtypeStruct((NC, NP, DEGW), jnp.float32),
    mesh=_sc_mesh,
    scratch_types=[
        pltpu.VMEM((NCHUNK, CHUNK), jnp.int32),
        pltpu.VMEM((CHUNK, DEGW), jnp.float32),
        pltpu.SemaphoreType.DMA,
        pltpu.VMEM_SHARED((NP, DEGW), jnp.float32),
    ],
    compiler_params=pltpu.CompilerParams(use_tc_tiling_on_sc=False),
)(_deg_body)


def _prop_body(table_hbm, src_hbm, dst_hbm, zeros_hbm, part_out,
               src_v, dst_v, buf, sem, accum):
    c = lax.axis_index("c")
    s = lax.axis_index("s")
    rows = pl.ds(s * RPT, RPT)
    pltpu.sync_copy(zeros_hbm.at[rows], accum.at[rows])
    pltpu.sync_copy(src_hbm.at[_wid(c, s)], src_v)
    pltpu.sync_copy(dst_hbm.at[_wid(c, s)], dst_v)
    plsc.subcore_barrier()

    def body(j, carry):
        pltpu.async_copy(table_hbm.at[src_v.at[j]], buf, sem).wait()
        pltpu.sync_copy(buf, accum.at[dst_v.at[j]], add=True)
        return carry

    lax.fori_loop(0, NCHUNK, body, 0)
    plsc.subcore_barrier()
    pltpu.sync_copy(accum.at[rows], part_out.at[c, rows])


def _prop_kernel(d):
    return functools.partial(
        pl.kernel,
        out_type=jax.ShapeDtypeStruct((NC, NP, d), jnp.float32),
        mesh=_sc_mesh,
        scratch_types=[
            pltpu.VMEM((NCHUNK, CHUNK), jnp.int32),
            pltpu.VMEM((NCHUNK, CHUNK), jnp.int32),
            pltpu.VMEM((CHUNK, d), jnp.float32),
            pltpu.SemaphoreType.DMA,
            pltpu.VMEM_SHARED((NP, d), jnp.float32),
        ],
        compiler_params=pltpu.CompilerParams(use_tc_tiling_on_sc=False),
    )(_prop_body)


_prop128 = _prop_kernel(DF)
_prop64 = _prop_kernel(DC)


def _dinv(dp_ref):
    return lax.rsqrt(dp_ref[0] + dp_ref[1] + 1.0)[:, 0:1]


def _prescale_body(dp_ref, x_ref, o_ref):
    o_ref[...] = x_ref[...] * _dinv(dp_ref)


def _mid_body(dp_ref, pp_ref, xs_ref, w1_ref, b1_ref, w2_ref, o_ref):
    dinv = _dinv(dp_ref)
    p = (pp_ref[0] + pp_ref[1] + xs_ref[...]) * dinv
    h = jnp.dot(p, w1_ref[...], preferred_element_type=jnp.float32)
    h = jnp.maximum(h + b1_ref[0:1, :], 0.0)
    q = jnp.dot(h, w2_ref[...], preferred_element_type=jnp.float32)
    o_ref[...] = q * dinv


def _final_body(dp_ref, rp_ref, qs_ref, b2_ref, o_ref):
    o_ref[...] = (rp_ref[0] + rp_ref[1] + qs_ref[...]) * _dinv(dp_ref) + b2_ref[0:1, :]


_BN = 1024


def _dp_spec():
    return pl.BlockSpec((NC, _BN, DEGW), lambda i: (0, i, 0))


def _row_spec(d):
    return pl.BlockSpec((_BN, d), lambda i: (i, 0))


def _part_spec(d):
    return pl.BlockSpec((NC, _BN, d), lambda i: (0, i, 0))


def _full_spec(r, d):
    return pl.BlockSpec((r, d), lambda i: (0, 0))


_prescale = pl.pallas_call(
    _prescale_body,
    grid=(NP // _BN,),
    in_specs=[_dp_spec(), _row_spec(DF)],
    out_specs=_row_spec(DF),
    out_shape=jax.ShapeDtypeStruct((NP, DF), jnp.float32),
)

_mid = pl.pallas_call(
    _mid_body,
    grid=(NP // _BN,),
    in_specs=[_dp_spec(), _part_spec(DF), _row_spec(DF),
              _full_spec(DF, DH), _full_spec(8, DH), _full_spec(DH, DC)],
    out_specs=_row_spec(DC),
    out_shape=jax.ShapeDtypeStruct((NP, DC), jnp.float32),
)

_final = pl.pallas_call(
    _final_body,
    grid=(NP // _BN,),
    in_specs=[_dp_spec(), _part_spec(DC), _row_spec(DC), _full_spec(8, DC)],
    out_specs=_row_spec(DC),
    out_shape=jax.ShapeDtypeStruct((NP, DC), jnp.float32),
)


@jax.jit
def kernel(x, edge_index, W1, b1, W2, b2):
    src = edge_index[0].astype(jnp.int32).reshape(NW, NCHUNK, CHUNK)
    dst = edge_index[1].astype(jnp.int32).reshape(NW, NCHUNK, CHUNK)

    xp = jnp.zeros((NP, DF), jnp.float32).at[:N].set(x)
    ones_rows = jnp.ones((CHUNK, DEGW), jnp.float32)
    zeros_deg = jnp.zeros((NP, DEGW), jnp.float32)
    zeros_f = jnp.zeros((NP, DF), jnp.float32)
    zeros_c = jnp.zeros((NP, DC), jnp.float32)

    deg_part = _deg_kernel(dst, ones_rows, zeros_deg)
    xs = _prescale(deg_part, xp)
    p_part = _prop128(xs, src, dst, zeros_f)
    qs = _mid(deg_part, p_part, xs, W1,
              jnp.broadcast_to(b1, (8, DH)), W2)
    r_part = _prop64(qs, src, dst, zeros_c)
    out = _final(deg_part, r_part, qs, jnp.broadcast_to(b2, (8, DC)))
    return out[:N]


# speedup vs baseline: 27.2305x; 1.0206x over previous
import functools

import jax
import jax.numpy as jnp
from jax import lax
from jax.experimental import pallas as pl
from jax.experimental.pallas import tpu as pltpu
from jax.experimental.pallas import tpu_sc as plsc

N = 10000
E = 320000
DF = 128
DH = 256
DC = 64

NC = 2
NS = 16
NW = NC * NS
EPW = E // NW
CHUNK = 125
NCHUNK = EPW // CHUNK
RPT = N // NS
DEGW = 8

_sc_mesh = plsc.VectorSubcoreMesh(core_axis_name="c", subcore_axis_name="s")


def _wid(c, s):
    return c * NS + s


def _deg_body(dst_hbm, ones_hbm, zeros_hbm, deg_out, idx_v, ones_v, sem, accum):
    c = lax.axis_index("c")
    s = lax.axis_index("s")
    rows = pl.ds(s * RPT, RPT)
    pltpu.sync_copy(zeros_hbm.at[rows], accum.at[rows])
    pltpu.sync_copy(ones_hbm, ones_v)
    pltpu.sync_copy(dst_hbm.at[_wid(c, s)], idx_v)
    plsc.subcore_barrier()

    def body(j, carry):
        pltpu.sync_copy(ones_v, accum.at[idx_v.at[j]], add=True)
        return carry

    lax.fori_loop(0, NCHUNK, body, 0)
    plsc.subcore_barrier()
    pltpu.sync_copy(accum.at[rows], deg_out.at[c, rows])


_deg_kernel = functools.partial(
    pl.kernel,
    out_type=jax.ShapeDtypeStruct((NC, N, DEGW), jnp.float32),
    mesh=_sc_mesh,
    scratch_types=[
        pltpu.VMEM((NCHUNK, CHUNK), jnp.int32),
        pltpu.VMEM((CHUNK, DEGW), jnp.float32),
        pltpu.SemaphoreType.DMA,
        pltpu.VMEM_SHARED((N, DEGW), jnp.float32),
    ],
    compiler_params=pltpu.CompilerParams(use_tc_tiling_on_sc=False),
)(_deg_body)


def _prop_body(table_hbm, src_hbm, dst_hbm, zeros_hbm, part_out,
               src_v, dst_v, buf_a, buf_b, sem_a, sem_b, accum):
    c = lax.axis_index("c")
    s = lax.axis_index("s")
    rows = pl.ds(s * RPT, RPT)
    pltpu.sync_copy(zeros_hbm.at[rows], accum.at[rows])
    pltpu.sync_copy(src_hbm.at[_wid(c, s)], src_v)
    pltpu.sync_copy(dst_hbm.at[_wid(c, s)], dst_v)
    plsc.subcore_barrier()

    def gather(j, buf, sem):
        pltpu.async_copy(table_hbm.at[src_v.at[j]], buf, sem)

    def gwait(j, buf, sem):
        pltpu.make_async_copy(table_hbm.at[src_v.at[j]], buf, sem).wait()

    gather(0, buf_a, sem_a)

    def body(j2, carry):
        j = 2 * j2
        gwait(j, buf_a, sem_a)
        gather(j + 1, buf_b, sem_b)
        pltpu.sync_copy(buf_a, accum.at[dst_v.at[j]], add=True)
        gwait(j + 1, buf_b, sem_b)
        gather(jnp.minimum(j + 2, NCHUNK - 1), buf_a, sem_a)
        pltpu.sync_copy(buf_b, accum.at[dst_v.at[j + 1]], add=True)
        return carry

    lax.fori_loop(0, NCHUNK // 2, body, 0)
    gwait(NCHUNK - 1, buf_a, sem_a)
    plsc.subcore_barrier()
    pltpu.sync_copy(accum.at[rows], part_out.at[c, rows])


def _prop_kernel(d):
    return functools.partial(
        pl.kernel,
        out_type=jax.ShapeDtypeStruct((NC, N, d), jnp.float32),
        mesh=_sc_mesh,
        scratch_types=[
            pltpu.VMEM((NCHUNK, CHUNK), jnp.int32),
            pltpu.VMEM((NCHUNK, CHUNK), jnp.int32),
            pltpu.VMEM((CHUNK, d), jnp.float32),
            pltpu.VMEM((CHUNK, d), jnp.float32),
            pltpu.SemaphoreType.DMA,
            pltpu.SemaphoreType.DMA,
            pltpu.VMEM_SHARED((N, d), jnp.float32),
        ],
        compiler_params=pltpu.CompilerParams(use_tc_tiling_on_sc=False),
    )(_prop_body)


_prop64 = _prop_kernel(DC)


def _dinv(dp_ref):
    return lax.rsqrt(dp_ref[0] + dp_ref[1] + 1.0)[:, 0:1]


def _prescale_body(dp_ref, x_ref, lo_ref, hi_ref):
    xs = x_ref[...] * _dinv(dp_ref)
    lo_ref[...] = xs[:, :DC]
    hi_ref[...] = xs[:, DC:]


def _mid_body(dp_ref, plo_ref, phi_ref, xlo_ref, xhi_ref, w1_ref, b1_ref, w2_ref, o_ref):
    dinv = _dinv(dp_ref)
    p_lo = plo_ref[0] + plo_ref[1] + xlo_ref[...]
    p_hi = phi_ref[0] + phi_ref[1] + xhi_ref[...]
    p = jnp.concatenate([p_lo, p_hi], axis=1) * dinv
    h = jnp.dot(p, w1_ref[...], preferred_element_type=jnp.float32)
    h = jnp.maximum(h + b1_ref[0:1, :], 0.0)
    q = jnp.dot(h, w2_ref[...], preferred_element_type=jnp.float32)
    o_ref[...] = q * dinv


def _final_body(dp_ref, rp_ref, qs_ref, b2_ref, o_ref):
    o_ref[...] = (rp_ref[0] + rp_ref[1] + qs_ref[...]) * _dinv(dp_ref) + b2_ref[0:1, :]


_BN = 1000


def _dp_spec():
    return pl.BlockSpec((NC, _BN, DEGW), lambda i: (0, i, 0))


def _row_spec(d):
    return pl.BlockSpec((_BN, d), lambda i: (i, 0))


def _part_spec(d):
    return pl.BlockSpec((NC, _BN, d), lambda i: (0, i, 0))


def _full_spec(r, d):
    return pl.BlockSpec((r, d), lambda i: (0, 0))


_prescale = pl.pallas_call(
    _prescale_body,
    grid=(N // _BN,),
    in_specs=[_dp_spec(), _row_spec(DF)],
    out_specs=[_row_spec(DC), _row_spec(DC)],
    out_shape=[jax.ShapeDtypeStruct((N, DC), jnp.float32),
               jax.ShapeDtypeStruct((N, DC), jnp.float32)],
)

_mid = pl.pallas_call(
    _mid_body,
    grid=(N // _BN,),
    in_specs=[_dp_spec(), _part_spec(DC), _part_spec(DC),
              _row_spec(DC), _row_spec(DC),
              _full_spec(DF, DH), _full_spec(8, DH), _full_spec(DH, DC)],
    out_specs=_row_spec(DC),
    out_shape=jax.ShapeDtypeStruct((N, DC), jnp.float32),
)

_final = pl.pallas_call(
    _final_body,
    grid=(N // _BN,),
    in_specs=[_dp_spec(), _part_spec(DC), _row_spec(DC), _full_spec(8, DC)],
    out_specs=_row_spec(DC),
    out_shape=jax.ShapeDtypeStruct((N, DC), jnp.float32),
)


@jax.jit
def kernel(x, edge_index, W1, b1, W2, b2):
    src = edge_index[0].astype(jnp.int32).reshape(NW, NCHUNK, CHUNK)
    dst = edge_index[1].astype(jnp.int32).reshape(NW, NCHUNK, CHUNK)

    ones_rows = jnp.ones((CHUNK, DEGW), jnp.float32)
    zeros_deg = jnp.zeros((N, DEGW), jnp.float32)
    zeros_c = jnp.zeros((N, DC), jnp.float32)

    deg_part = _deg_kernel(dst, ones_rows, zeros_deg)
    x_lo, x_hi = _prescale(deg_part, x)
    p_lo = _prop64(x_lo, src, dst, zeros_c)
    p_hi = _prop64(x_hi, src, dst, zeros_c)
    qs = _mid(deg_part, p_lo, p_hi, x_lo, x_hi, W1,
              jnp.broadcast_to(b1, (8, DH)), W2)
    r_part = _prop64(qs, src, dst, zeros_c)
    return _final(deg_part, r_part, qs, jnp.broadcast_to(b2, (8, DC)))


# speedup vs baseline: 27.4566x; 1.0083x over previous
import functools

import jax
import jax.numpy as jnp
from jax import lax
from jax.experimental import pallas as pl
from jax.experimental.pallas import tpu as pltpu
from jax.experimental.pallas import tpu_sc as plsc

N = 10000
E = 320000
DF = 128
DH = 256
DC = 64

NC = 2
NS = 16
NW = NC * NS
EPW = E // NW
CHUNK = 125
NCHUNK = EPW // CHUNK
RPT = N // NS
DEGW = 4

_sc_mesh = plsc.VectorSubcoreMesh(core_axis_name="c", subcore_axis_name="s")


def _wid(c, s):
    return c * NS + s


def _deg_body(edges_hbm, ones_hbm, zeros_hbm, deg_out, idx_v, ones_v, sem, accum):
    c = lax.axis_index("c")
    s = lax.axis_index("s")
    rows = pl.ds(s * RPT, RPT)
    pltpu.sync_copy(zeros_hbm.at[rows], accum.at[rows])
    pltpu.sync_copy(ones_hbm, ones_v)
    pltpu.sync_copy(edges_hbm.at[1, _wid(c, s)], idx_v)
    plsc.subcore_barrier()

    def body(j, carry):
        pltpu.sync_copy(ones_v, accum.at[idx_v.at[j]], add=True)
        return carry

    lax.fori_loop(0, NCHUNK, body, 0)
    plsc.subcore_barrier()
    pltpu.sync_copy(accum.at[rows], deg_out.at[c, rows])


_deg_kernel = functools.partial(
    pl.kernel,
    out_type=jax.ShapeDtypeStruct((NC, N, DEGW), jnp.float32),
    mesh=_sc_mesh,
    scratch_types=[
        pltpu.VMEM((NCHUNK, CHUNK), jnp.int32),
        pltpu.VMEM((CHUNK, DEGW), jnp.float32),
        pltpu.SemaphoreType.DMA,
        pltpu.VMEM_SHARED((N, DEGW), jnp.float32),
    ],
    compiler_params=pltpu.CompilerParams(use_tc_tiling_on_sc=False),
)(_deg_body)


def _prop_body(table_hbm, edges_hbm, zeros_hbm, part_out,
               src_v, dst_v, buf_a, buf_b, sem_a, sem_b, accum):
    c = lax.axis_index("c")
    s = lax.axis_index("s")
    rows = pl.ds(s * RPT, RPT)
    pltpu.sync_copy(zeros_hbm.at[rows], accum.at[rows])
    pltpu.sync_copy(edges_hbm.at[0, _wid(c, s)], src_v)
    pltpu.sync_copy(edges_hbm.at[1, _wid(c, s)], dst_v)
    plsc.subcore_barrier()

    def gather(j, buf, sem):
        pltpu.async_copy(table_hbm.at[src_v.at[j]], buf, sem)

    def gwait(j, buf, sem):
        pltpu.make_async_copy(table_hbm.at[src_v.at[j]], buf, sem).wait()

    gather(0, buf_a, sem_a)

    def body(j2, carry):
        j = 2 * j2
        gwait(j, buf_a, sem_a)
        gather(j + 1, buf_b, sem_b)
        pltpu.sync_copy(buf_a, accum.at[dst_v.at[j]], add=True)
        gwait(j + 1, buf_b, sem_b)
        gather(jnp.minimum(j + 2, NCHUNK - 1), buf_a, sem_a)
        pltpu.sync_copy(buf_b, accum.at[dst_v.at[j + 1]], add=True)
        return carry

    lax.fori_loop(0, NCHUNK // 2, body, 0)
    gwait(NCHUNK - 1, buf_a, sem_a)
    plsc.subcore_barrier()
    pltpu.sync_copy(accum.at[rows], part_out.at[c, rows])


def _prop_kernel(d):
    return functools.partial(
        pl.kernel,
        out_type=jax.ShapeDtypeStruct((NC, N, d), jnp.float32),
        mesh=_sc_mesh,
        scratch_types=[
            pltpu.VMEM((NCHUNK, CHUNK), jnp.int32),
            pltpu.VMEM((NCHUNK, CHUNK), jnp.int32),
            pltpu.VMEM((CHUNK, d), jnp.float32),
            pltpu.VMEM((CHUNK, d), jnp.float32),
            pltpu.SemaphoreType.DMA,
            pltpu.SemaphoreType.DMA,
            pltpu.VMEM_SHARED((N, d), jnp.float32),
        ],
        compiler_params=pltpu.CompilerParams(use_tc_tiling_on_sc=False),
    )(_prop_body)


_prop64 = _prop_kernel(DC)


def _dinv(dp_ref):
    return lax.rsqrt(dp_ref[0] + dp_ref[1] + 1.0)[:, 0:1]


def _prescale_body(dp_ref, x_ref, lo_ref, hi_ref):
    xs = x_ref[...] * _dinv(dp_ref)
    lo_ref[...] = xs[:, :DC]
    hi_ref[...] = xs[:, DC:]


def _mid_body(dp_ref, plo_ref, phi_ref, xlo_ref, xhi_ref, w1_ref, b1_ref, w2_ref, o_ref):
    dinv = _dinv(dp_ref)
    p_lo = plo_ref[0] + plo_ref[1] + xlo_ref[...]
    p_hi = phi_ref[0] + phi_ref[1] + xhi_ref[...]
    p = jnp.concatenate([p_lo, p_hi], axis=1) * dinv
    h = jnp.dot(p, w1_ref[...], preferred_element_type=jnp.float32)
    h = jnp.maximum(h + b1_ref[0:1, :], 0.0)
    q = jnp.dot(h, w2_ref[...], preferred_element_type=jnp.float32)
    o_ref[...] = q * dinv


def _final_body(dp_ref, rp_ref, qs_ref, b2_ref, o_ref):
    o_ref[...] = (rp_ref[0] + rp_ref[1] + qs_ref[...]) * _dinv(dp_ref) + b2_ref[0:1, :]


_BN = 1000


def _dp_spec():
    return pl.BlockSpec((NC, _BN, DEGW), lambda i: (0, i, 0))


def _row_spec(d):
    return pl.BlockSpec((_BN, d), lambda i: (i, 0))


def _part_spec(d):
    return pl.BlockSpec((NC, _BN, d), lambda i: (0, i, 0))


def _full_spec(r, d):
    return pl.BlockSpec((r, d), lambda i: (0, 0))


_prescale = pl.pallas_call(
    _prescale_body,
    grid=(N // _BN,),
    in_specs=[_dp_spec(), _row_spec(DF)],
    out_specs=[_row_spec(DC), _row_spec(DC)],
    out_shape=[jax.ShapeDtypeStruct((N, DC), jnp.float32),
               jax.ShapeDtypeStruct((N, DC), jnp.float32)],
)

_mid = pl.pallas_call(
    _mid_body,
    grid=(N // _BN,),
    in_specs=[_dp_spec(), _part_spec(DC), _part_spec(DC),
              _row_spec(DC), _row_spec(DC),
              _full_spec(DF, DH), _full_spec(8, DH), _full_spec(DH, DC)],
    out_specs=_row_spec(DC),
    out_shape=jax.ShapeDtypeStruct((N, DC), jnp.float32),
)

_final = pl.pallas_call(
    _final_body,
    grid=(N // _BN,),
    in_specs=[_dp_spec(), _part_spec(DC), _row_spec(DC), _full_spec(8, DC)],
    out_specs=_row_spec(DC),
    out_shape=jax.ShapeDtypeStruct((N, DC), jnp.float32),
)


@jax.jit
def kernel(x, edge_index, W1, b1, W2, b2):
    edges = edge_index.astype(jnp.int32).reshape(2, NW, NCHUNK, CHUNK)

    ones_rows = jnp.ones((CHUNK, DEGW), jnp.float32)
    zeros_deg = jnp.zeros((N, DEGW), jnp.float32)
    zeros_c = jnp.zeros((N, DC), jnp.float32)

    deg_part = _deg_kernel(edges, ones_rows, zeros_deg)
    x_lo, x_hi = _prescale(deg_part, x)
    p_lo = _prop64(x_lo, edges, zeros_c)
    p_hi = _prop64(x_hi, edges, zeros_c)
    qs = _mid(deg_part, p_lo, p_hi, x_lo, x_hi, W1,
              jnp.broadcast_to(b1, (8, DH)), W2)
    r_part = _prop64(qs, edges, zeros_c)
    return _final(deg_part, r_part, qs, jnp.broadcast_to(b2, (8, DC)))


# speedup vs baseline: 27.8353x; 1.0138x over previous
import functools

import jax
import jax.numpy as jnp
from jax import lax
from jax.experimental import pallas as pl
from jax.experimental.pallas import tpu as pltpu
from jax.experimental.pallas import tpu_sc as plsc

N = 10000
E = 320000
DF = 128
DH = 256
DC = 64

NC = 2
NS = 16
NW = NC * NS
EPW = E // NW
CHUNK = 125
NCHUNK = EPW // CHUNK
RPT = N // NS
DEGW = 8

_sc_mesh = plsc.VectorSubcoreMesh(core_axis_name="c", subcore_axis_name="s")


def _wid(c, s):
    return c * NS + s


def _deg_body(edges_hbm, ones_hbm, zeros_hbm, deg_out, idx_v, ones_v, sem, accum):
    c = lax.axis_index("c")
    s = lax.axis_index("s")
    rows = pl.ds(s * RPT, RPT)
    pltpu.sync_copy(zeros_hbm.at[rows], accum.at[rows])
    pltpu.sync_copy(ones_hbm, ones_v)
    pltpu.sync_copy(edges_hbm.at[1, _wid(c, s)], idx_v)
    plsc.subcore_barrier()

    def body(j, carry):
        pltpu.sync_copy(ones_v, accum.at[idx_v.at[j]], add=True)
        return carry

    lax.fori_loop(0, NCHUNK, body, 0)
    plsc.subcore_barrier()
    pltpu.sync_copy(accum.at[rows], deg_out.at[c, rows])


_deg_kernel = functools.partial(
    pl.kernel,
    out_type=jax.ShapeDtypeStruct((NC, N, DEGW), jnp.float32),
    mesh=_sc_mesh,
    scratch_types=[
        pltpu.VMEM((NCHUNK, CHUNK), jnp.int32),
        pltpu.VMEM((CHUNK, DEGW), jnp.float32),
        pltpu.SemaphoreType.DMA,
        pltpu.VMEM_SHARED((N, DEGW), jnp.float32),
    ],
    compiler_params=pltpu.CompilerParams(use_tc_tiling_on_sc=False),
)(_deg_body)


def _prop_body(table_hbm, edges_hbm, zeros_hbm, part_out,
               src_v, dst_v, buf_a, buf_b, sem_a, sem_b, accum):
    c = lax.axis_index("c")
    s = lax.axis_index("s")
    rows = pl.ds(s * RPT, RPT)
    pltpu.sync_copy(zeros_hbm.at[rows], accum.at[rows])
    pltpu.sync_copy(edges_hbm.at[0, _wid(c, s)], src_v)
    pltpu.sync_copy(edges_hbm.at[1, _wid(c, s)], dst_v)
    plsc.subcore_barrier()

    def gather(j, buf, sem):
        pltpu.async_copy(table_hbm.at[src_v.at[j]], buf, sem)

    def gwait(j, buf, sem):
        pltpu.make_async_copy(table_hbm.at[src_v.at[j]], buf, sem).wait()

    gather(0, buf_a, sem_a)

    def body(j2, carry):
        j = 2 * j2
        gwait(j, buf_a, sem_a)
        gather(j + 1, buf_b, sem_b)
        pltpu.sync_copy(buf_a, accum.at[dst_v.at[j]], add=True)
        gwait(j + 1, buf_b, sem_b)
        gather(jnp.minimum(j + 2, NCHUNK - 1), buf_a, sem_a)
        pltpu.sync_copy(buf_b, accum.at[dst_v.at[j + 1]], add=True)
        return carry

    lax.fori_loop(0, NCHUNK // 2, body, 0)
    gwait(NCHUNK - 1, buf_a, sem_a)
    plsc.subcore_barrier()
    pltpu.sync_copy(accum.at[rows], part_out.at[c, rows])


def _prop_kernel(d):
    return functools.partial(
        pl.kernel,
        out_type=jax.ShapeDtypeStruct((NC, N, d), jnp.float32),
        mesh=_sc_mesh,
        scratch_types=[
            pltpu.VMEM((NCHUNK, CHUNK), jnp.int32),
            pltpu.VMEM((NCHUNK, CHUNK), jnp.int32),
            pltpu.VMEM((CHUNK, d), jnp.float32),
            pltpu.VMEM((CHUNK, d), jnp.float32),
            pltpu.SemaphoreType.DMA,
            pltpu.SemaphoreType.DMA,
            pltpu.VMEM_SHARED((N, d), jnp.float32),
        ],
        compiler_params=pltpu.CompilerParams(use_tc_tiling_on_sc=False),
    )(_prop_body)


_prop64 = _prop_kernel(DC)


def _dinv(dp_ref):
    return lax.rsqrt(dp_ref[0] + dp_ref[1] + 1.0)[:, 0:1]


def _prescale_body(dp_ref, x_ref, lo_ref, hi_ref):
    xs = x_ref[...] * _dinv(dp_ref)
    lo_ref[...] = xs[:, :DC]
    hi_ref[...] = xs[:, DC:]


def _mid_body(dp_ref, plo_ref, phi_ref, xlo_ref, xhi_ref, w1_ref, b1_ref, w2_ref, o_ref):
    dinv = _dinv(dp_ref)
    p_lo = plo_ref[0] + plo_ref[1] + xlo_ref[...]
    p_hi = phi_ref[0] + phi_ref[1] + xhi_ref[...]
    p = jnp.concatenate([p_lo, p_hi], axis=1) * dinv
    h = jnp.dot(p, w1_ref[...], preferred_element_type=jnp.float32)
    h = jnp.maximum(h + b1_ref[0:1, :], 0.0)
    q = jnp.dot(h, w2_ref[...], preferred_element_type=jnp.float32)
    o_ref[...] = q * dinv


def _final_body(dp_ref, rp_ref, qs_ref, b2_ref, o_ref):
    o_ref[...] = (rp_ref[0] + rp_ref[1] + qs_ref[...]) * _dinv(dp_ref) + b2_ref[0:1, :]


_BN = 1000


def _dp_spec():
    return pl.BlockSpec((NC, _BN, DEGW), lambda i: (0, i, 0))


def _row_spec(d):
    return pl.BlockSpec((_BN, d), lambda i: (i, 0))


def _part_spec(d):
    return pl.BlockSpec((NC, _BN, d), lambda i: (0, i, 0))


def _full_spec(r, d):
    return pl.BlockSpec((r, d), lambda i: (0, 0))


_prescale = pl.pallas_call(
    _prescale_body,
    grid=(N // _BN,),
    in_specs=[_dp_spec(), _row_spec(DF)],
    out_specs=[_row_spec(DC), _row_spec(DC)],
    out_shape=[jax.ShapeDtypeStruct((N, DC), jnp.float32),
               jax.ShapeDtypeStruct((N, DC), jnp.float32)],
)

_mid = pl.pallas_call(
    _mid_body,
    grid=(N // _BN,),
    in_specs=[_dp_spec(), _part_spec(DC), _part_spec(DC),
              _row_spec(DC), _row_spec(DC),
              _full_spec(DF, DH), _full_spec(8, DH), _full_spec(DH, DC)],
    out_specs=_row_spec(DC),
    out_shape=jax.ShapeDtypeStruct((N, DC), jnp.float32),
)

_final = pl.pallas_call(
    _final_body,
    grid=(N // _BN,),
    in_specs=[_dp_spec(), _part_spec(DC), _row_spec(DC), _full_spec(8, DC)],
    out_specs=_row_spec(DC),
    out_shape=jax.ShapeDtypeStruct((N, DC), jnp.float32),
)


@jax.jit
def kernel(x, edge_index, W1, b1, W2, b2):
    edges = edge_index.astype(jnp.int32).reshape(2, NW, NCHUNK, CHUNK)

    ones_rows = jnp.ones((CHUNK, DEGW), jnp.float32)
    zeros_deg = jnp.zeros((N, DEGW), jnp.float32)
    zeros_c = jnp.zeros((N, DC), jnp.float32)

    deg_part = _deg_kernel(edges, ones_rows, zeros_deg)
    x_lo, x_hi = _prescale(deg_part, x)
    p_lo = _prop64(x_lo, edges, zeros_c)
    p_hi = _prop64(x_hi, edges, zeros_c)
    qs = _mid(deg_part, p_lo, p_hi, x_lo, x_hi, W1,
              jnp.broadcast_to(b1, (8, DH)), W2)
    r_part = _prop64(qs, edges, zeros_c)
    return _final(deg_part, r_part, qs, jnp.broadcast_to(b2, (8, DC)))


# speedup vs baseline: 38.6403x; 1.3882x over previous
import functools

import jax
import jax.numpy as jnp
from jax import lax
from jax.experimental import pallas as pl
from jax.experimental.pallas import tpu as pltpu
from jax.experimental.pallas import tpu_sc as plsc

N = 10000
E = 320000
DF = 128
DH = 256
DC = 64

NC = 2
NS = 16
NW = NC * NS
EPW = E // NW
CHUNK = 125
NCHUNK = EPW // CHUNK
RPT = N // NS
DEGW = 8

_sc_mesh = plsc.VectorSubcoreMesh(core_axis_name="c", subcore_axis_name="s")


def _wid(c, s):
    return c * NS + s


def _deg_body(edges_hbm, ones_hbm, zeros_hbm, deg_out, idx_v, ones_v, sem, accum):
    c = lax.axis_index("c")
    s = lax.axis_index("s")
    rows = pl.ds(s * RPT, RPT)
    pltpu.sync_copy(zeros_hbm.at[rows], accum.at[rows])
    pltpu.sync_copy(ones_hbm, ones_v)
    pltpu.sync_copy(edges_hbm.at[1, _wid(c, s)], idx_v)
    plsc.subcore_barrier()

    def body(j, carry):
        pltpu.sync_copy(ones_v, accum.at[idx_v.at[j]], add=True)
        return carry

    lax.fori_loop(0, NCHUNK, body, 0)
    plsc.subcore_barrier()
    pltpu.sync_copy(accum.at[rows], deg_out.at[c, rows])


_deg_kernel = functools.partial(
    pl.kernel,
    out_type=jax.ShapeDtypeStruct((NC, N, DEGW), jnp.float32),
    mesh=_sc_mesh,
    scratch_types=[
        pltpu.VMEM((NCHUNK, CHUNK), jnp.int32),
        pltpu.VMEM((CHUNK, DEGW), jnp.float32),
        pltpu.SemaphoreType.DMA,
        pltpu.VMEM_SHARED((N, DEGW), jnp.float32),
    ],
    compiler_params=pltpu.CompilerParams(use_tc_tiling_on_sc=False),
)(_deg_body)


def _prop_body(table_hbm, edges_hbm, zeros_hbm, part_out,
               src_v, dst_v, buf_a, buf_b, buf_c, buf_d,
               sem_a, sem_b, sem_c, sem_d, accum):
    c = lax.axis_index("c")
    s = lax.axis_index("s")
    rows = pl.ds(s * RPT, RPT)
    pltpu.sync_copy(zeros_hbm.at[rows], accum.at[rows])
    pltpu.sync_copy(edges_hbm.at[0, _wid(c, s)], src_v)
    pltpu.sync_copy(edges_hbm.at[1, _wid(c, s)], dst_v)
    plsc.subcore_barrier()

    def gather(j, buf, sem):
        pltpu.async_copy(table_hbm.at[src_v.at[j]], buf, sem)

    def gwait(j, buf, sem):
        pltpu.make_async_copy(table_hbm.at[src_v.at[j]], buf, sem).wait()

    bufs = (buf_a, buf_b, buf_c, buf_d)
    sems = (sem_a, sem_b, sem_c, sem_d)
    for k in range(4):
        gather(k, bufs[k], sems[k])

    def body(j4, carry):
        j = 4 * j4
        for k in range(4):
            gwait(j + k, bufs[k], sems[k])
            gather(jnp.minimum(j + k + 4, NCHUNK - 1), bufs[k], sems[k])
            pltpu.sync_copy(bufs[k], accum.at[dst_v.at[j + k]], add=True)
        return carry

    lax.fori_loop(0, NCHUNK // 4, body, 0)
    for k in range(4):
        gwait(NCHUNK - 1, bufs[k], sems[k])
    plsc.subcore_barrier()
    pltpu.sync_copy(accum.at[rows], part_out.at[c, rows])


def _prop_kernel(d):
    return functools.partial(
        pl.kernel,
        out_type=jax.ShapeDtypeStruct((NC, N, d), jnp.float32),
        mesh=_sc_mesh,
        scratch_types=[
            pltpu.VMEM((NCHUNK, CHUNK), jnp.int32),
            pltpu.VMEM((NCHUNK, CHUNK), jnp.int32),
            pltpu.VMEM((CHUNK, d), jnp.float32),
            pltpu.VMEM((CHUNK, d), jnp.float32),
            pltpu.VMEM((CHUNK, d), jnp.float32),
            pltpu.VMEM((CHUNK, d), jnp.float32),
            pltpu.SemaphoreType.DMA,
            pltpu.SemaphoreType.DMA,
            pltpu.SemaphoreType.DMA,
            pltpu.SemaphoreType.DMA,
            pltpu.VMEM_SHARED((N, d), jnp.float32),
        ],
        compiler_params=pltpu.CompilerParams(use_tc_tiling_on_sc=False),
    )(_prop_body)


_prop64 = _prop_kernel(DC)


def _dinv(dp_ref):
    return lax.rsqrt(dp_ref[0] + dp_ref[1] + 1.0)[:, 0:1]


def _prescale_body(dp_ref, x_ref, lo_ref, hi_ref):
    xs = x_ref[...] * _dinv(dp_ref)
    lo_ref[...] = xs[:, :DC]
    hi_ref[...] = xs[:, DC:]


def _mid_body(dp_ref, plo_ref, phi_ref, xlo_ref, xhi_ref, w1_ref, b1_ref, w2_ref, o_ref):
    dinv = _dinv(dp_ref)
    p_lo = plo_ref[0] + plo_ref[1] + xlo_ref[...]
    p_hi = phi_ref[0] + phi_ref[1] + xhi_ref[...]
    p = jnp.concatenate([p_lo, p_hi], axis=1) * dinv
    h = jnp.dot(p, w1_ref[...], preferred_element_type=jnp.float32)
    h = jnp.maximum(h + b1_ref[0:1, :], 0.0)
    q = jnp.dot(h, w2_ref[...], preferred_element_type=jnp.float32)
    o_ref[...] = q * dinv


def _final_body(dp_ref, rp_ref, qs_ref, b2_ref, o_ref):
    o_ref[...] = (rp_ref[0] + rp_ref[1] + qs_ref[...]) * _dinv(dp_ref) + b2_ref[0:1, :]


_BN = 1000


def _dp_spec():
    return pl.BlockSpec((NC, _BN, DEGW), lambda i: (0, i, 0))


def _row_spec(d):
    return pl.BlockSpec((_BN, d), lambda i: (i, 0))


def _part_spec(d):
    return pl.BlockSpec((NC, _BN, d), lambda i: (0, i, 0))


def _full_spec(r, d):
    return pl.BlockSpec((r, d), lambda i: (0, 0))


_prescale = pl.pallas_call(
    _prescale_body,
    grid=(N // _BN,),
    in_specs=[_dp_spec(), _row_spec(DF)],
    out_specs=[_row_spec(DC), _row_spec(DC)],
    out_shape=[jax.ShapeDtypeStruct((N, DC), jnp.float32),
               jax.ShapeDtypeStruct((N, DC), jnp.float32)],
)

_mid = pl.pallas_call(
    _mid_body,
    grid=(N // _BN,),
    in_specs=[_dp_spec(), _part_spec(DC), _part_spec(DC),
              _row_spec(DC), _row_spec(DC),
              _full_spec(DF, DH), _full_spec(8, DH), _full_spec(DH, DC)],
    out_specs=_row_spec(DC),
    out_shape=jax.ShapeDtypeStruct((N, DC), jnp.float32),
)

_final = pl.pallas_call(
    _final_body,
    grid=(N // _BN,),
    in_specs=[_dp_spec(), _part_spec(DC), _row_spec(DC), _full_spec(8, DC)],
    out_specs=_row_spec(DC),
    out_shape=jax.ShapeDtypeStruct((N, DC), jnp.float32),
)


@jax.jit
def kernel(x, edge_index, W1, b1, W2, b2):
    edges = edge_index.astype(jnp.int32).reshape(2, NW, NCHUNK, CHUNK)

    ones_rows = jnp.ones((CHUNK, DEGW), jnp.float32)
    zeros_deg = jnp.zeros((N, DEGW), jnp.float32)
    zeros_c = jnp.zeros((N, DC), jnp.float32)

    deg_part = _deg_kernel(edges, ones_rows, zeros_deg)
    x_lo, x_hi = _prescale(deg_part, x)
    p_lo = _prop64(x_lo, edges, zeros_c)
    p_hi = _prop64(x_hi, edges, zeros_c)
    qs = _mid(deg_part, p_lo, p_hi, x_lo, x_hi, W1,
              jnp.broadcast_to(b1, (8, DH)), W2)
    r_part = _prop64(qs, edges, zeros_c)
    return _final(deg_part, r_part, qs, jnp.broadcast_to(b2, (8, DC)))
